# baseline (speedup 1.0000x reference)
import jax
import jax.numpy as jnp
from jax.experimental import pallas as pl

_GRID = 128
_PQ_GRID = 512


def _copy_body(*refs):
    n = len(refs) // 2
    for src, dst in zip(refs[:n], refs[n:]):
        dst[...] = src[...]


def _stream(specs_vals, grid):
    vals = [v for v, _ in specs_vals]
    specs = [s for _, s in specs_vals]
    return pl.pallas_call(
        _copy_body,
        grid=(grid,),
        in_specs=specs,
        out_specs=specs,
        out_shape=tuple(jax.ShapeDtypeStruct(v.shape, v.dtype) for v in vals),
    )(*vals)


def kernel(sensor_data, state, force, pq_samples, p, q, future_state,
           p_smooth, q_smooth, cost, z_mu, z_var, sensor_data_pred,
           count, positions,
           state_buffer, force_buffer, sensor_data_buffer,
           sensor_data_pred_buffer, pq_samples_buffer, p_buffer, q_buffer,
           p_buffer_smooth, q_buffer_smooth, cost_buffer,
           future_state_buffer, z_mu_buffer, z_var_buffer, iter_buffer):
    del positions
    del state_buffer, force_buffer, sensor_data_buffer
    del sensor_data_pred_buffer, pq_samples_buffer, p_buffer, q_buffer
    del p_buffer_smooth, q_buffer_smooth, cost_buffer
    del future_state_buffer, z_mu_buffer, z_var_buffer

    count = count.astype(iter_buffer.dtype)
    b = sensor_data.shape[0]
    bm = b // _GRID

    def native3d(v):
        return v, pl.BlockSpec((bm,) + v.shape[1:], lambda i: (i, 0, 0))

    def flat128(v):
        m = v.size // 128
        if m % (8 * _GRID) == 0:
            return (v.reshape(m, 128),
                    pl.BlockSpec((m // _GRID, 128), lambda i: (i, 0)))
        return v.reshape(m, 128), pl.BlockSpec((m, 128), lambda i: (0, 0))

    main = [
        native3d(sensor_data),
        flat128(state),
        flat128(force),
        flat128(p),
        flat128(q),
        native3d(future_state),
        flat128(p_smooth),
        flat128(q_smooth),
        flat128(cost),
        flat128(z_mu),
        flat128(z_var),
        native3d(sensor_data_pred),
        flat128(count),
    ]
    main.append(flat128(pq_samples))
    (sd_o, st_o, f_o, p_o, q_o, fs_o, ps_o, qs_o, c_o, zm_o, zv_o,
     sp_o, ct_o, pq_o) = _stream(main, _GRID)

    return (sd_o, st_o.reshape(state.shape), f_o.reshape(force.shape),
            pq_o.reshape(pq_samples.shape), p_o.reshape(p.shape), q_o.reshape(q.shape), fs_o,
            ps_o.reshape(p_smooth.shape), qs_o.reshape(q_smooth.shape),
            c_o.reshape(cost.shape), zm_o.reshape(z_mu.shape),
            zv_o.reshape(z_var.shape), sp_o, ct_o.reshape(count.shape))


# speedup vs baseline: 1.7663x; 1.7663x over previous
import jax
import jax.numpy as jnp
from jax.experimental import pallas as pl

_GRID = 64
_PQ_GRID = 512


def _copy_body(*refs):
    n = len(refs) // 2
    for src, dst in zip(refs[:n], refs[n:]):
        dst[...] = src[...]


def _stream(specs_vals, grid):
    vals = [v for v, _ in specs_vals]
    specs = [s for _, s in specs_vals]
    return pl.pallas_call(
        _copy_body,
        grid=(grid,),
        in_specs=specs,
        out_specs=specs,
        out_shape=tuple(jax.ShapeDtypeStruct(v.shape, v.dtype) for v in vals),
    )(*vals)


def kernel(sensor_data, state, force, pq_samples, p, q, future_state,
           p_smooth, q_smooth, cost, z_mu, z_var, sensor_data_pred,
           count, positions,
           state_buffer, force_buffer, sensor_data_buffer,
           sensor_data_pred_buffer, pq_samples_buffer, p_buffer, q_buffer,
           p_buffer_smooth, q_buffer_smooth, cost_buffer,
           future_state_buffer, z_mu_buffer, z_var_buffer, iter_buffer):
    del positions
    del state_buffer, force_buffer, sensor_data_buffer
    del sensor_data_pred_buffer, pq_samples_buffer, p_buffer, q_buffer
    del p_buffer_smooth, q_buffer_smooth, cost_buffer
    del future_state_buffer, z_mu_buffer, z_var_buffer

    count = count.astype(iter_buffer.dtype)
    b = sensor_data.shape[0]
    bm = b // _GRID

    def native3d(v):
        return v, pl.BlockSpec((bm,) + v.shape[1:], lambda i: (i, 0, 0))

    def flat128(v):
        m = v.size // 128
        if m % (8 * _GRID) == 0:
            return (v.reshape(m, 128),
                    pl.BlockSpec((m // _GRID, 128), lambda i: (i, 0)))
        return v.reshape(m, 128), pl.BlockSpec((m, 128), lambda i: (0, 0))

    main = [
        native3d(sensor_data),
        flat128(state),
        flat128(force),
        flat128(p),
        flat128(q),
        native3d(future_state),
        flat128(p_smooth),
        flat128(q_smooth),
        flat128(cost),
        flat128(z_mu),
        flat128(z_var),
        native3d(sensor_data_pred),
        flat128(count),
    ]
    main.append(native3d(pq_samples))
    (sd_o, st_o, f_o, p_o, q_o, fs_o, ps_o, qs_o, c_o, zm_o, zv_o,
     sp_o, ct_o, pq_o) = _stream(main, _GRID)

    return (sd_o, st_o.reshape(state.shape), f_o.reshape(force.shape),
            pq_o, p_o.reshape(p.shape), q_o.reshape(q.shape), fs_o,
            ps_o.reshape(p_smooth.shape), qs_o.reshape(q_smooth.shape),
            c_o.reshape(cost.shape), zm_o.reshape(z_mu.shape),
            zv_o.reshape(z_var.shape), sp_o, ct_o.reshape(count.shape))
